# Initial kernel scaffold; baseline (speedup 1.0000x reference)
import jax
import jax.numpy as jnp
from jax.experimental import pallas as pl


def kernel(x, selection_index, selection_probabilities, W_bank, b_bank):
    raise NotImplementedError("write your pallas kernel here")


# speedup vs baseline: 3.2348x; 3.2348x over previous
import functools

import jax
import jax.numpy as jnp
from jax.experimental import pallas as pl
from jax.experimental.pallas import tpu as pltpu

IN_F = 1024
OUT_F = 1024
S_T = 512


def _vm_kernel(idx_ref, p_ref, x_ref, w0_ref, w1_ref, b0_ref, b1_ref, out_ref):
    b = pl.program_id(0)
    p0 = p_ref[b, 0]
    p1 = p_ref[b, 1]
    w = p0 * w0_ref[0] + p1 * w1_ref[0]
    bias = p0 * b0_ref[0, 0] + p1 * b1_ref[0, 0]
    acc = jnp.dot(x_ref[0], w, preferred_element_type=jnp.float32)
    out_ref[0] = acc + bias[None, :]


@jax.jit
def kernel(x, selection_index, selection_probabilities, W_bank, b_bank):
    B, S, _ = x.shape
    grid = (B, S // S_T)
    b_bank3 = b_bank[:, None, :]

    grid_spec = pltpu.PrefetchScalarGridSpec(
        num_scalar_prefetch=1,
        grid=grid,
        in_specs=[
            pl.BlockSpec(memory_space=pltpu.SMEM),
            pl.BlockSpec((1, S_T, IN_F), lambda b, s, idx: (b, s, 0)),
            pl.BlockSpec((1, IN_F, OUT_F), lambda b, s, idx: (idx[b, 0], 0, 0)),
            pl.BlockSpec((1, IN_F, OUT_F), lambda b, s, idx: (idx[b, 1], 0, 0)),
            pl.BlockSpec((1, 1, OUT_F), lambda b, s, idx: (idx[b, 0], 0, 0)),
            pl.BlockSpec((1, 1, OUT_F), lambda b, s, idx: (idx[b, 1], 0, 0)),
        ],
        out_specs=pl.BlockSpec((1, S_T, OUT_F), lambda b, s, idx: (b, s, 0)),
    )

    out = pl.pallas_call(
        _vm_kernel,
        grid_spec=grid_spec,
        out_shape=jax.ShapeDtypeStruct((B, S, OUT_F), jnp.float32),
        compiler_params=pltpu.CompilerParams(
            dimension_semantics=("arbitrary", "arbitrary"),
        ),
    )(selection_index, selection_probabilities, x, W_bank, W_bank, b_bank3, b_bank3)
    return out
